# Initial kernel scaffold; baseline (speedup 1.0000x reference)
import jax
import jax.numpy as jnp
from jax.experimental import pallas as pl


def kernel(x, adj, basis1, comb1, basis2, comb2, fc1_w, fc1_b, fc2_w, fc2_b):
    raise NotImplementedError("write your pallas kernel here")


# speedup vs baseline: 1.0152x; 1.0152x over previous
import functools

import jax
import jax.numpy as jnp
from jax.experimental import pallas as pl
from jax.experimental.pallas import tpu as pltpu

_N = 4096
_NHID = 512
_SUPPORT = 4
_NB = 2
_NCLASS = 4
_PADC = 128

_TM = 512
_TK = 1024


def _xb_body(x_ref, b_ref, out_ref):
    out_ref[...] = jnp.dot(
        x_ref[...], b_ref[...], preferred_element_type=jnp.float32
    ).astype(jnp.bfloat16)


def _combine(comb_ref, adj_ref, col):
    a = adj_ref[...]
    acc = a[0] * comb_ref[0, col]
    for r in range(1, _SUPPORT):
        acc = acc + a[r] * comb_ref[r, col]
    return acc.astype(jnp.bfloat16)


def _adj_matmul_step(comb_ref, adj_ref, xb_ref, acc_ref):
    a0 = _combine(comb_ref, adj_ref, 0)
    a1 = _combine(comb_ref, adj_ref, 1)
    part = jnp.dot(a0, xb_ref[:, :_NHID], preferred_element_type=jnp.float32)
    part = part + jnp.dot(
        a1, xb_ref[:, _NHID:], preferred_element_type=jnp.float32
    )
    k = pl.program_id(1)

    @pl.when(k == 0)
    def _init():
        acc_ref[...] = part

    @pl.when(k > 0)
    def _accum():
        acc_ref[...] = acc_ref[...] + part


def _layer1_body(comb_ref, adj_ref, xb_ref, bnext_ref, out_ref, acc_ref):
    _adj_matmul_step(comb_ref, adj_ref, xb_ref, acc_ref)

    @pl.when(pl.program_id(1) == pl.num_programs(1) - 1)
    def _epilogue():
        h = jnp.maximum(acc_ref[...], 0.0)
        out_ref[...] = jnp.dot(
            h, bnext_ref[...], preferred_element_type=jnp.float32
        ).astype(jnp.bfloat16)


def _layer2_body(comb_ref, adj_ref, xb_ref, fc1w_ref, fc1b_ref, fc2w_ref,
                 fc2b_ref, out_ref, acc_ref):
    _adj_matmul_step(comb_ref, adj_ref, xb_ref, acc_ref)

    @pl.when(pl.program_id(1) == pl.num_programs(1) - 1)
    def _epilogue():
        h2 = jnp.maximum(acc_ref[...], 0.0)
        h3 = jnp.maximum(
            jnp.dot(h2, fc1w_ref[...], preferred_element_type=jnp.float32)
            + fc1b_ref[...],
            0.0,
        )
        logits = (
            jnp.dot(h3, fc2w_ref[...], preferred_element_type=jnp.float32)
            + fc2b_ref[...]
        )
        m = jnp.max(logits, axis=-1, keepdims=True)
        lse = m + jnp.log(
            jnp.sum(jnp.exp(logits - m), axis=-1, keepdims=True)
        )
        out_ref[...] = logits - lse


def _layer_call(body, extra_inputs, extra_specs, comb, adj, xb, out_cols,
                out_dtype):
    grid = (_N // _TM, _N // _TK)
    return pl.pallas_call(
        body,
        grid=grid,
        in_specs=[
            pl.BlockSpec(memory_space=pltpu.SMEM),
            pl.BlockSpec((_SUPPORT, _TM, _TK), lambda i, k: (0, i, k)),
            pl.BlockSpec((_TK, _NB * _NHID), lambda i, k: (k, 0)),
        ] + extra_specs,
        out_specs=pl.BlockSpec((_TM, out_cols), lambda i, k: (i, 0)),
        out_shape=jax.ShapeDtypeStruct((_N, out_cols), out_dtype),
        scratch_shapes=[pltpu.VMEM((_TM, _NHID), jnp.float32)],
        compiler_params=pltpu.CompilerParams(
            dimension_semantics=("parallel", "arbitrary"),
        ),
    )(comb, adj, xb, *extra_inputs)


def kernel(x, adj, basis1, comb1, basis2, comb2, fc1_w, fc1_b, fc2_w, fc2_b):
    bc1 = jnp.concatenate([basis1[0], basis1[1]], axis=1)
    bc2 = jnp.concatenate([basis2[0], basis2[1]], axis=1)
    fc2_wp = jnp.zeros((_NHID, _PADC), jnp.float32).at[:, :_NCLASS].set(fc2_w)
    fc2_bp = jnp.full((1, _PADC), -1e30, jnp.float32).at[0, :_NCLASS].set(fc2_b)
    fc1_b2 = fc1_b.reshape(1, _NHID)

    xb1 = pl.pallas_call(
        _xb_body,
        grid=(_N // _TM,),
        in_specs=[
            pl.BlockSpec((_TM, _NHID), lambda i: (i, 0)),
            pl.BlockSpec((_NHID, _NB * _NHID), lambda i: (0, 0)),
        ],
        out_specs=pl.BlockSpec((_TM, _NB * _NHID), lambda i: (i, 0)),
        out_shape=jax.ShapeDtypeStruct((_N, _NB * _NHID), jnp.bfloat16),
    )(x, bc1)

    const_spec = lambda shape: pl.BlockSpec(shape, lambda i, k: (0, 0))

    xb2 = _layer_call(
        _layer1_body,
        [bc2],
        [const_spec((_NHID, _NB * _NHID))],
        comb1, adj, xb1,
        _NB * _NHID, jnp.bfloat16,
    )

    logp = _layer_call(
        _layer2_body,
        [fc1_w, fc1_b2, fc2_wp, fc2_bp],
        [
            const_spec((_NHID, _NHID)),
            const_spec((1, _NHID)),
            const_spec((_NHID, _PADC)),
            const_spec((1, _PADC)),
        ],
        comb2, adj, xb2,
        _PADC, jnp.float32,
    )

    return logp[:, :_NCLASS]


# speedup vs baseline: 1.0687x; 1.0527x over previous
import jax
import jax.numpy as jnp
from jax.experimental import pallas as pl
from jax.experimental.pallas import tpu as pltpu

_N = 4096
_NHID = 512
_SUPPORT = 4
_NB = 2
_NCLASS = 4
_PADC = 128

_TM = 512
_TK = 1024


def _xb_body(x_ref, b_ref, out_ref):
    out_ref[...] = jnp.dot(
        x_ref[...], b_ref[...], preferred_element_type=jnp.float32
    ).astype(jnp.bfloat16)


def _combine(comb_ref, a, col):
    acc = a[0] * comb_ref[0, col]
    for r in range(1, _SUPPORT):
        acc = acc + a[r] * comb_ref[r, col]
    return acc.astype(jnp.bfloat16)


def _accumulate(part, acc_ref):
    k = pl.program_id(1)

    @pl.when(k == 0)
    def _init():
        acc_ref[...] = part

    @pl.when(k > 0)
    def _accum():
        acc_ref[...] = acc_ref[...] + part


def _layer1_body(comb_ref, adj_ref, xb_ref, bnext_ref, a2_ref, out_ref,
                 acc_ref):
    a = adj_ref[...]
    a0 = _combine(comb_ref, a, 0)
    a1 = _combine(comb_ref, a, 1)
    a2_ref[0] = _combine(comb_ref, a, 2)
    a2_ref[1] = _combine(comb_ref, a, 3)
    part = jnp.dot(a0, xb_ref[:, :_NHID], preferred_element_type=jnp.float32)
    part = part + jnp.dot(
        a1, xb_ref[:, _NHID:], preferred_element_type=jnp.float32
    )
    _accumulate(part, acc_ref)

    @pl.when(pl.program_id(1) == pl.num_programs(1) - 1)
    def _epilogue():
        h = jnp.maximum(acc_ref[...], 0.0)
        out_ref[...] = jnp.dot(
            h, bnext_ref[...], preferred_element_type=jnp.float32
        ).astype(jnp.bfloat16)


def _layer2_body(a_ref, xb_ref, fc1w_ref, fc1b_ref, fc2w_ref, fc2b_ref,
                 out_ref, acc_ref):
    part = jnp.dot(
        a_ref[0], xb_ref[:, :_NHID], preferred_element_type=jnp.float32
    )
    part = part + jnp.dot(
        a_ref[1], xb_ref[:, _NHID:], preferred_element_type=jnp.float32
    )
    _accumulate(part, acc_ref)

    @pl.when(pl.program_id(1) == pl.num_programs(1) - 1)
    def _epilogue():
        h2 = jnp.maximum(acc_ref[...], 0.0)
        h3 = jnp.maximum(
            jnp.dot(h2, fc1w_ref[...], preferred_element_type=jnp.float32)
            + fc1b_ref[...],
            0.0,
        )
        logits = (
            jnp.dot(h3, fc2w_ref[...], preferred_element_type=jnp.float32)
            + fc2b_ref[...]
        )
        m = jnp.max(logits, axis=-1, keepdims=True)
        lse = m + jnp.log(
            jnp.sum(jnp.exp(logits - m), axis=-1, keepdims=True)
        )
        out_ref[...] = logits - lse


def kernel(x, adj, basis1, comb1, basis2, comb2, fc1_w, fc1_b, fc2_w, fc2_b):
    bc1 = jnp.concatenate([basis1[0], basis1[1]], axis=1)
    bc2 = jnp.concatenate([basis2[0], basis2[1]], axis=1)
    comb12 = jnp.concatenate([comb1, comb2], axis=1)
    fc2_wp = jnp.zeros((_NHID, _PADC), jnp.float32).at[:, :_NCLASS].set(fc2_w)
    fc2_bp = jnp.full((1, _PADC), -1e30, jnp.float32).at[0, :_NCLASS].set(fc2_b)
    fc1_b2 = fc1_b.reshape(1, _NHID)

    xb1 = pl.pallas_call(
        _xb_body,
        grid=(_N // _TM,),
        in_specs=[
            pl.BlockSpec((_TM, _NHID), lambda i: (i, 0)),
            pl.BlockSpec((_NHID, _NB * _NHID), lambda i: (0, 0)),
        ],
        out_specs=pl.BlockSpec((_TM, _NB * _NHID), lambda i: (i, 0)),
        out_shape=jax.ShapeDtypeStruct((_N, _NB * _NHID), jnp.bfloat16),
    )(x, bc1)

    grid = (_N // _TM, _N // _TK)
    const_spec = lambda shape: pl.BlockSpec(shape, lambda i, k: (0, 0))

    a2c, xb2 = pl.pallas_call(
        _layer1_body,
        grid=grid,
        in_specs=[
            pl.BlockSpec(memory_space=pltpu.SMEM),
            pl.BlockSpec((_SUPPORT, _TM, _TK), lambda i, k: (0, i, k)),
            pl.BlockSpec((_TK, _NB * _NHID), lambda i, k: (k, 0)),
            const_spec((_NHID, _NB * _NHID)),
        ],
        out_specs=[
            pl.BlockSpec((_NB, _TM, _TK), lambda i, k: (0, i, k)),
            pl.BlockSpec((_TM, _NB * _NHID), lambda i, k: (i, 0)),
        ],
        out_shape=[
            jax.ShapeDtypeStruct((_NB, _N, _N), jnp.bfloat16),
            jax.ShapeDtypeStruct((_N, _NB * _NHID), jnp.bfloat16),
        ],
        scratch_shapes=[pltpu.VMEM((_TM, _NHID), jnp.float32)],
        compiler_params=pltpu.CompilerParams(
            dimension_semantics=("parallel", "arbitrary"),
        ),
    )(comb12, adj, xb1, bc2)

    logp = pl.pallas_call(
        _layer2_body,
        grid=grid,
        in_specs=[
            pl.BlockSpec((_NB, _TM, _TK), lambda i, k: (0, i, k)),
            pl.BlockSpec((_TK, _NB * _NHID), lambda i, k: (k, 0)),
            const_spec((_NHID, _NHID)),
            const_spec((1, _NHID)),
            const_spec((_NHID, _PADC)),
            const_spec((1, _PADC)),
        ],
        out_specs=pl.BlockSpec((_TM, _PADC), lambda i, k: (i, 0)),
        out_shape=jax.ShapeDtypeStruct((_N, _PADC), jnp.float32),
        scratch_shapes=[pltpu.VMEM((_TM, _NHID), jnp.float32)],
        compiler_params=pltpu.CompilerParams(
            dimension_semantics=("parallel", "arbitrary"),
        ),
    )(a2c, xb2, fc1_w, fc1_b2, fc2_wp, fc2_bp)

    return logp[:, :_NCLASS]


# speedup vs baseline: 1.2212x; 1.1427x over previous
import jax
import jax.numpy as jnp
from jax.experimental import pallas as pl
from jax.experimental.pallas import tpu as pltpu

_N = 4096
_NHID = 512
_SUPPORT = 4
_NB = 2
_NCLASS = 4
_PADC = 128

_TM = 512
_TK = 1024


def _xb_body(x_ref, b_ref, out_ref):
    out_ref[...] = jnp.dot(
        x_ref[...], b_ref[...], preferred_element_type=jnp.float32
    ).astype(jnp.bfloat16)


def _combine(comb_ref, a, col):
    acc = a[0] * comb_ref[0, col]
    for r in range(1, _SUPPORT):
        acc = acc + a[r] * comb_ref[r, col]
    return acc.astype(jnp.bfloat16)


def _accumulate(part, acc_ref):
    k = pl.program_id(1)

    @pl.when(k == 0)
    def _init():
        acc_ref[...] = part

    @pl.when(k > 0)
    def _accum():
        acc_ref[...] = acc_ref[...] + part


def _layer1_body(comb_ref, adj_ref, xb_ref, bnext_ref, a2_ref, out_ref,
                 acc_ref):
    a = adj_ref[...]
    a0 = _combine(comb_ref, a, 0)
    a1 = _combine(comb_ref, a, 1)
    a2_ref[0] = _combine(comb_ref, a, 2)
    a2_ref[1] = _combine(comb_ref, a, 3)
    xk = _TK * pl.program_id(1)
    part = jnp.dot(
        a0, xb_ref[pl.ds(xk, _TK), :_NHID], preferred_element_type=jnp.float32
    )
    part = part + jnp.dot(
        a1, xb_ref[pl.ds(xk, _TK), _NHID:], preferred_element_type=jnp.float32
    )
    _accumulate(part, acc_ref)

    @pl.when(pl.program_id(1) == pl.num_programs(1) - 1)
    def _epilogue():
        h = jnp.maximum(acc_ref[...], 0.0)
        out_ref[...] = jnp.dot(
            h, bnext_ref[...], preferred_element_type=jnp.float32
        ).astype(jnp.bfloat16)


def _layer2_body(a_ref, xb_ref, fc1w_ref, fc1b_ref, fc2w_ref, fc2b_ref,
                 out_ref):
    h2 = jnp.dot(
        a_ref[0], xb_ref[:, :_NHID], preferred_element_type=jnp.float32
    )
    h2 = h2 + jnp.dot(
        a_ref[1], xb_ref[:, _NHID:], preferred_element_type=jnp.float32
    )
    h2 = jnp.maximum(h2, 0.0)
    h3 = jnp.maximum(
        jnp.dot(h2, fc1w_ref[...], preferred_element_type=jnp.float32)
        + fc1b_ref[...],
        0.0,
    )
    logits = (
        jnp.dot(h3, fc2w_ref[...], preferred_element_type=jnp.float32)
        + fc2b_ref[...]
    )
    m = jnp.max(logits, axis=-1, keepdims=True)
    lse = m + jnp.log(
        jnp.sum(jnp.exp(logits - m), axis=-1, keepdims=True)
    )
    out_ref[...] = logits - lse


def kernel(x, adj, basis1, comb1, basis2, comb2, fc1_w, fc1_b, fc2_w, fc2_b):
    bc1 = jnp.concatenate([basis1[0], basis1[1]], axis=1)
    bc2 = jnp.concatenate([basis2[0], basis2[1]], axis=1)
    comb12 = jnp.concatenate([comb1, comb2], axis=1)
    fc2_wp = jnp.zeros((_NHID, _PADC), jnp.float32).at[:, :_NCLASS].set(fc2_w)
    fc2_bp = jnp.full((1, _PADC), -1e30, jnp.float32).at[0, :_NCLASS].set(fc2_b)
    fc1_b2 = fc1_b.reshape(1, _NHID)

    xb1 = pl.pallas_call(
        _xb_body,
        grid=(_N // _TM,),
        in_specs=[
            pl.BlockSpec((_TM, _NHID), lambda i: (i, 0)),
            pl.BlockSpec((_NHID, _NB * _NHID), lambda i: (0, 0)),
        ],
        out_specs=pl.BlockSpec((_TM, _NB * _NHID), lambda i: (i, 0)),
        out_shape=jax.ShapeDtypeStruct((_N, _NB * _NHID), jnp.bfloat16),
    )(x, bc1)

    grid = (_N // _TM, _N // _TK)
    const_spec = lambda shape: pl.BlockSpec(shape, lambda i, k: (0, 0))

    a2c, xb2 = pl.pallas_call(
        _layer1_body,
        grid=grid,
        in_specs=[
            pl.BlockSpec(memory_space=pltpu.SMEM),
            pl.BlockSpec((_SUPPORT, _TM, _TK), lambda i, k: (0, i, k)),
            pl.BlockSpec((_N, _NB * _NHID), lambda i, k: (0, 0)),
            const_spec((_NHID, _NB * _NHID)),
        ],
        out_specs=[
            pl.BlockSpec((_NB, _TM, _TK), lambda i, k: (0, i, k)),
            pl.BlockSpec((_TM, _NB * _NHID), lambda i, k: (i, 0)),
        ],
        out_shape=[
            jax.ShapeDtypeStruct((_NB, _N, _N), jnp.bfloat16),
            jax.ShapeDtypeStruct((_N, _NB * _NHID), jnp.bfloat16),
        ],
        scratch_shapes=[pltpu.VMEM((_TM, _NHID), jnp.float32)],
        compiler_params=pltpu.CompilerParams(
            dimension_semantics=("parallel", "arbitrary"),
        ),
    )(comb12, adj, xb1, bc2)

    logp = pl.pallas_call(
        _layer2_body,
        grid=(_N // _TM,),
        in_specs=[
            pl.BlockSpec((_NB, _TM, _N), lambda i: (0, i, 0)),
            pl.BlockSpec((_N, _NB * _NHID), lambda i: (0, 0)),
            pl.BlockSpec((_NHID, _NHID), lambda i: (0, 0)),
            pl.BlockSpec((1, _NHID), lambda i: (0, 0)),
            pl.BlockSpec((_NHID, _PADC), lambda i: (0, 0)),
            pl.BlockSpec((1, _PADC), lambda i: (0, 0)),
        ],
        out_specs=pl.BlockSpec((_TM, _PADC), lambda i: (i, 0)),
        out_shape=jax.ShapeDtypeStruct((_N, _PADC), jnp.float32),
        compiler_params=pltpu.CompilerParams(
            dimension_semantics=("arbitrary",),
        ),
    )(a2c, xb2, fc1_w, fc1_b2, fc2_wp, fc2_bp)

    return logp[:, :_NCLASS]
